# baseline (speedup 1.0000x reference)
import functools

import jax
import jax.numpy as jnp
from jax import lax
from jax.experimental import pallas as pl
from jax.experimental.pallas import tpu as pltpu
from jax.experimental.pallas import tpu_sc as plsc

K = 8192
D = 32
N = 8192
TT = 512
CK = 2048
CC = 0.25


def _vq_tc_body(x_ref, w_ref, idx_ref, loss_ref):
    i = pl.program_id(0)
    x = x_ref[...]
    a = jnp.sum(x * x, axis=1)
    xb = (-2.0 * x).astype(jnp.bfloat16)

    def step(k, carry):
        mind, mini = carry
        w = w_ref[pl.ds(k * CK, CK), :]
        hi = w.astype(jnp.bfloat16)
        lo = (w - hi.astype(jnp.float32)).astype(jnp.bfloat16)
        dn = (((1,), (1,)), ((), ()))
        m = (lax.dot_general(xb, hi, dn, preferred_element_type=jnp.float32)
             + lax.dot_general(xb, lo, dn, preferred_element_type=jnp.float32))
        c = jnp.sum(w * w, axis=1)[None, :]
        t = m + c
        tmin = jnp.min(t, axis=1)
        iota = lax.broadcasted_iota(jnp.int32, (TT, CK), 1) + k * CK
        tidx = jnp.min(jnp.where(t == tmin[:, None], iota, K), axis=1)
        better = tmin < mind
        return (jnp.where(better, tmin, mind), jnp.where(better, tidx, mini))

    mind0 = jnp.full((TT,), jnp.inf, dtype=jnp.float32)
    mini0 = jnp.zeros((TT,), dtype=jnp.int32)
    mind, mini = lax.fori_loop(0, K // CK, step, (mind0, mini0))
    idx_ref[...] = mini

    @pl.when(i == 0)
    def _():
        loss_ref[0, 0] = 0.0
    loss_ref[0, 0] += jnp.sum(a + mind)


def _vq_argmin(flat, W):
    return pl.pallas_call(
        _vq_tc_body,
        grid=(N // TT,),
        in_specs=[
            pl.BlockSpec((TT, D), lambda i: (i, 0)),
            pl.BlockSpec((K, D), lambda i: (0, 0)),
        ],
        out_specs=[
            pl.BlockSpec((TT,), lambda i: (i,)),
            pl.BlockSpec((1, 1), lambda i: (0, 0), memory_space=pltpu.SMEM),
        ],
        out_shape=[
            jax.ShapeDtypeStruct((N,), jnp.int32),
            jax.ShapeDtypeStruct((1, 1), jnp.float32),
        ],
    )(flat, W)


def _sc_gather(table, idx):
    info = plsc.get_sparse_core_info()
    nw = info.num_cores * info.num_subcores
    b_per_w = N // nw
    mesh = plsc.VectorSubcoreMesh(core_axis_name="c", subcore_axis_name="s")

    @functools.partial(
        pl.kernel, mesh=mesh,
        compiler_params=pltpu.CompilerParams(use_tc_tiling_on_sc=False),
        out_type=jax.ShapeDtypeStruct((N, D), jnp.float32),
        scratch_types=[
            pltpu.VMEM((b_per_w,), jnp.int32),
            pltpu.VMEM((b_per_w, D), jnp.float32),
            pltpu.SemaphoreType.DMA,
        ],
    )
    def k(idx_hbm, table_hbm, out_hbm, idx_v, rows_v, sem):
        wid = lax.axis_index("s") * info.num_cores + lax.axis_index("c")
        base = wid * b_per_w
        pltpu.sync_copy(idx_hbm.at[pl.ds(base, b_per_w)], idx_v)
        pltpu.async_copy(table_hbm.at[idx_v], rows_v, sem).wait()
        pltpu.sync_copy(rows_v, out_hbm.at[pl.ds(base, b_per_w)])

    return k(idx, table)


def kernel(inputs, W):
    x = jnp.transpose(inputs, (0, 2, 3, 1))
    flat = x.reshape(N, D)
    idx, loss_sum = _vq_argmin(flat, W)
    q_flat = _sc_gather(W, idx)
    quantized = q_flat.reshape(x.shape)
    encodings = idx.reshape(x.shape[0:3])
    loss = (1.0 + CC) * (loss_sum[0, 0] / (N * D))
    quantized_st = x + (quantized - x)
    out = jnp.transpose(quantized_st, (0, 3, 1, 2))
    nll = jnp.ones((1,), dtype=jnp.float32)
    return (out, encodings, loss, nll)


# speedup vs baseline: 1.2185x; 1.2185x over previous
import functools

import jax
import jax.numpy as jnp
from jax import lax
from jax.experimental import pallas as pl
from jax.experimental.pallas import tpu as pltpu
from jax.experimental.pallas import tpu_sc as plsc

K = 8192
D = 32
N = 8192
TT = 512
CK = 2048
CC = 0.25


def _vq_tc_body(x_ref, w_ref, idx_ref, loss_ref):
    i = pl.program_id(0)
    x = x_ref[...]
    a = jnp.sum(x * x, axis=1)
    xb = -2.0 * x

    def step(k, carry):
        mind, mini = carry
        w = w_ref[pl.ds(k * CK, CK), :]
        m = lax.dot_general(xb, w, (((1,), (1,)), ((), ())),
                            preferred_element_type=jnp.float32)
        c = jnp.sum(w * w, axis=1)[None, :]
        t = m + c
        tmin = jnp.min(t, axis=1)
        iota = lax.broadcasted_iota(jnp.int32, (TT, CK), 1) + k * CK
        tidx = jnp.min(jnp.where(t == tmin[:, None], iota, K), axis=1)
        better = tmin < mind
        return (jnp.where(better, tmin, mind), jnp.where(better, tidx, mini))

    mind0 = jnp.full((TT,), jnp.inf, dtype=jnp.float32)
    mini0 = jnp.zeros((TT,), dtype=jnp.int32)
    mind, mini = lax.fori_loop(0, K // CK, step, (mind0, mini0))
    idx_ref[...] = mini

    @pl.when(i == 0)
    def _():
        loss_ref[0, 0] = 0.0
    loss_ref[0, 0] += jnp.sum(a + mind)


def _vq_argmin(flat, W):
    return pl.pallas_call(
        _vq_tc_body,
        grid=(N // TT,),
        in_specs=[
            pl.BlockSpec((TT, D), lambda i: (i, 0)),
            pl.BlockSpec((K, D), lambda i: (0, 0)),
        ],
        out_specs=[
            pl.BlockSpec((TT,), lambda i: (i,)),
            pl.BlockSpec((1, 1), lambda i: (0, 0), memory_space=pltpu.SMEM),
        ],
        out_shape=[
            jax.ShapeDtypeStruct((N,), jnp.int32),
            jax.ShapeDtypeStruct((1, 1), jnp.float32),
        ],
    )(flat, W)


def _sc_gather(table, idx):
    info = plsc.get_sparse_core_info()
    nw = info.num_cores * info.num_subcores
    b_per_w = N // nw
    mesh = plsc.VectorSubcoreMesh(core_axis_name="c", subcore_axis_name="s")

    @functools.partial(
        pl.kernel, mesh=mesh,
        compiler_params=pltpu.CompilerParams(use_tc_tiling_on_sc=False),
        out_type=jax.ShapeDtypeStruct((N, D), jnp.float32),
        scratch_types=[
            pltpu.VMEM((b_per_w,), jnp.int32),
            pltpu.VMEM((b_per_w, D), jnp.float32),
            pltpu.SemaphoreType.DMA,
        ],
    )
    def k(idx_hbm, table_hbm, out_hbm, idx_v, rows_v, sem):
        wid = lax.axis_index("s") * info.num_cores + lax.axis_index("c")
        base = wid * b_per_w
        pltpu.sync_copy(idx_hbm.at[pl.ds(base, b_per_w)], idx_v)
        pltpu.async_copy(table_hbm.at[idx_v], rows_v, sem).wait()
        pltpu.sync_copy(rows_v, out_hbm.at[pl.ds(base, b_per_w)])

    return k(idx, table)


def kernel(inputs, W):
    x = jnp.transpose(inputs, (0, 2, 3, 1))
    flat = x.reshape(N, D)
    idx, loss_sum = _vq_argmin(flat, W)
    q_flat = _sc_gather(W, idx)
    quantized = q_flat.reshape(x.shape)
    encodings = idx.reshape(x.shape[0:3])
    loss = (1.0 + CC) * (loss_sum[0, 0] / (N * D))
    quantized_st = x + (quantized - x)
    out = jnp.transpose(quantized_st, (0, 3, 1, 2))
    nll = jnp.ones((1,), dtype=jnp.float32)
    return (out, encodings, loss, nll)


# speedup vs baseline: 1.3096x; 1.0748x over previous
import functools

import jax
import jax.numpy as jnp
from jax import lax
from jax.experimental import pallas as pl
from jax.experimental.pallas import tpu as pltpu
from jax.experimental.pallas import tpu_sc as plsc

K = 8192
D = 32
N = 8192
TT = 1024
CK = 2048
CC = 0.25


def _vq_tc_body(x_ref, w_ref, idx_ref, loss_ref):
    i = pl.program_id(0)
    x = x_ref[...]
    a = jnp.sum(x * x, axis=1)
    xb = -2.0 * x

    def step(k, carry):
        mind, mini = carry
        w = w_ref[pl.ds(k * CK, CK), :]
        m = lax.dot_general(xb, w, (((1,), (1,)), ((), ())),
                            preferred_element_type=jnp.float32)
        c = jnp.sum(w * w, axis=1)[None, :]
        t = m + c
        tmin = jnp.min(t, axis=1)
        iota = lax.broadcasted_iota(jnp.int32, (TT, CK), 1) + k * CK
        tidx = jnp.min(jnp.where(t == tmin[:, None], iota, K), axis=1)
        better = tmin < mind
        return (jnp.where(better, tmin, mind), jnp.where(better, tidx, mini))

    mind0 = jnp.full((TT,), jnp.inf, dtype=jnp.float32)
    mini0 = jnp.zeros((TT,), dtype=jnp.int32)
    mind, mini = lax.fori_loop(0, K // CK, step, (mind0, mini0))
    idx_ref[...] = mini

    @pl.when(i == 0)
    def _():
        loss_ref[0, 0] = 0.0
    loss_ref[0, 0] += jnp.sum(a + mind)


def _vq_argmin(flat, W):
    return pl.pallas_call(
        _vq_tc_body,
        grid=(N // TT,),
        in_specs=[
            pl.BlockSpec((TT, D), lambda i: (i, 0)),
            pl.BlockSpec((K, D), lambda i: (0, 0)),
        ],
        out_specs=[
            pl.BlockSpec((TT,), lambda i: (i,)),
            pl.BlockSpec((1, 1), lambda i: (0, 0), memory_space=pltpu.SMEM),
        ],
        out_shape=[
            jax.ShapeDtypeStruct((N,), jnp.int32),
            jax.ShapeDtypeStruct((1, 1), jnp.float32),
        ],
    )(flat, W)


def _sc_gather(table, idx):
    info = plsc.get_sparse_core_info()
    nw = info.num_cores * info.num_subcores
    b_per_w = N // nw
    mesh = plsc.VectorSubcoreMesh(core_axis_name="c", subcore_axis_name="s")

    @functools.partial(
        pl.kernel, mesh=mesh,
        compiler_params=pltpu.CompilerParams(use_tc_tiling_on_sc=False),
        out_type=jax.ShapeDtypeStruct((N, D), jnp.float32),
        scratch_types=[
            pltpu.VMEM((b_per_w,), jnp.int32),
            pltpu.VMEM((b_per_w, D), jnp.float32),
            pltpu.SemaphoreType.DMA,
        ],
    )
    def k(idx_hbm, table_hbm, out_hbm, idx_v, rows_v, sem):
        wid = lax.axis_index("s") * info.num_cores + lax.axis_index("c")
        base = wid * b_per_w
        pltpu.sync_copy(idx_hbm.at[pl.ds(base, b_per_w)], idx_v)
        pltpu.async_copy(table_hbm.at[idx_v], rows_v, sem).wait()
        pltpu.sync_copy(rows_v, out_hbm.at[pl.ds(base, b_per_w)])

    return k(idx, table)


def kernel(inputs, W):
    x = jnp.transpose(inputs, (0, 2, 3, 1))
    flat = x.reshape(N, D)
    idx, loss_sum = _vq_argmin(flat, W)
    q_flat = _sc_gather(W, idx)
    quantized = q_flat.reshape(x.shape)
    encodings = idx.reshape(x.shape[0:3])
    loss = (1.0 + CC) * (loss_sum[0, 0] / (N * D))
    quantized_st = x + (quantized - x)
    out = jnp.transpose(quantized_st, (0, 3, 1, 2))
    nll = jnp.ones((1,), dtype=jnp.float32)
    return (out, encodings, loss, nll)


# speedup vs baseline: 1.3499x; 1.0307x over previous
import functools

import jax
import jax.numpy as jnp
from jax import lax
from jax.experimental import pallas as pl
from jax.experimental.pallas import tpu as pltpu
from jax.experimental.pallas import tpu_sc as plsc

K = 8192
D = 32
N = 8192
TT = 2048
CK = 2048
CC = 0.25


def _vq_tc_body(x_ref, w_ref, idx_ref, loss_ref):
    i = pl.program_id(0)
    x = x_ref[...]
    a = jnp.sum(x * x, axis=1)
    xb = -2.0 * x

    def step(k, carry):
        mind, mini = carry
        w = w_ref[pl.ds(k * CK, CK), :]
        m = lax.dot_general(xb, w, (((1,), (1,)), ((), ())),
                            preferred_element_type=jnp.float32)
        c = jnp.sum(w * w, axis=1)[None, :]
        t = m + c
        tmin = jnp.min(t, axis=1)
        iota = lax.broadcasted_iota(jnp.int32, (TT, CK), 1) + k * CK
        tidx = jnp.min(jnp.where(t == tmin[:, None], iota, K), axis=1)
        better = tmin < mind
        return (jnp.where(better, tmin, mind), jnp.where(better, tidx, mini))

    mind0 = jnp.full((TT,), jnp.inf, dtype=jnp.float32)
    mini0 = jnp.zeros((TT,), dtype=jnp.int32)
    mind, mini = lax.fori_loop(0, K // CK, step, (mind0, mini0))
    idx_ref[...] = mini

    @pl.when(i == 0)
    def _():
        loss_ref[0, 0] = 0.0
    loss_ref[0, 0] += jnp.sum(a + mind)


def _vq_argmin(flat, W):
    return pl.pallas_call(
        _vq_tc_body,
        grid=(N // TT,),
        in_specs=[
            pl.BlockSpec((TT, D), lambda i: (i, 0)),
            pl.BlockSpec((K, D), lambda i: (0, 0)),
        ],
        out_specs=[
            pl.BlockSpec((TT,), lambda i: (i,)),
            pl.BlockSpec((1, 1), lambda i: (0, 0), memory_space=pltpu.SMEM),
        ],
        out_shape=[
            jax.ShapeDtypeStruct((N,), jnp.int32),
            jax.ShapeDtypeStruct((1, 1), jnp.float32),
        ],
    )(flat, W)


def _sc_gather(table, idx):
    info = plsc.get_sparse_core_info()
    nw = info.num_cores * info.num_subcores
    b_per_w = N // nw
    mesh = plsc.VectorSubcoreMesh(core_axis_name="c", subcore_axis_name="s")

    @functools.partial(
        pl.kernel, mesh=mesh,
        compiler_params=pltpu.CompilerParams(use_tc_tiling_on_sc=False),
        out_type=jax.ShapeDtypeStruct((N, D), jnp.float32),
        scratch_types=[
            pltpu.VMEM((b_per_w,), jnp.int32),
            pltpu.VMEM((b_per_w, D), jnp.float32),
            pltpu.SemaphoreType.DMA,
        ],
    )
    def k(idx_hbm, table_hbm, out_hbm, idx_v, rows_v, sem):
        wid = lax.axis_index("s") * info.num_cores + lax.axis_index("c")
        base = wid * b_per_w
        pltpu.sync_copy(idx_hbm.at[pl.ds(base, b_per_w)], idx_v)
        pltpu.async_copy(table_hbm.at[idx_v], rows_v, sem).wait()
        pltpu.sync_copy(rows_v, out_hbm.at[pl.ds(base, b_per_w)])

    return k(idx, table)


def kernel(inputs, W):
    x = jnp.transpose(inputs, (0, 2, 3, 1))
    flat = x.reshape(N, D)
    idx, loss_sum = _vq_argmin(flat, W)
    q_flat = _sc_gather(W, idx)
    quantized = q_flat.reshape(x.shape)
    encodings = idx.reshape(x.shape[0:3])
    loss = (1.0 + CC) * (loss_sum[0, 0] / (N * D))
    quantized_st = x + (quantized - x)
    out = jnp.transpose(quantized_st, (0, 3, 1, 2))
    nll = jnp.ones((1,), dtype=jnp.float32)
    return (out, encodings, loss, nll)


# speedup vs baseline: 1.3671x; 1.0128x over previous
import functools

import jax
import jax.numpy as jnp
from jax import lax
from jax.experimental import pallas as pl
from jax.experimental.pallas import tpu as pltpu
from jax.experimental.pallas import tpu_sc as plsc

K = 8192
D = 32
N = 8192
TT = 4096
CK = 2048
CC = 0.25


def _vq_tc_body(x_ref, w_ref, idx_ref, loss_ref):
    i = pl.program_id(0)
    x = x_ref[...]
    a = jnp.sum(x * x, axis=1)
    xb = -2.0 * x

    def step(k, carry):
        mind, mini = carry
        w = w_ref[pl.ds(k * CK, CK), :]
        m = lax.dot_general(xb, w, (((1,), (1,)), ((), ())),
                            preferred_element_type=jnp.float32)
        c = jnp.sum(w * w, axis=1)[None, :]
        t = m + c
        tmin = jnp.min(t, axis=1)
        iota = lax.broadcasted_iota(jnp.int32, (TT, CK), 1) + k * CK
        tidx = jnp.min(jnp.where(t == tmin[:, None], iota, K), axis=1)
        better = tmin < mind
        return (jnp.where(better, tmin, mind), jnp.where(better, tidx, mini))

    mind0 = jnp.full((TT,), jnp.inf, dtype=jnp.float32)
    mini0 = jnp.zeros((TT,), dtype=jnp.int32)
    mind, mini = lax.fori_loop(0, K // CK, step, (mind0, mini0))
    idx_ref[...] = mini

    @pl.when(i == 0)
    def _():
        loss_ref[0, 0] = 0.0
    loss_ref[0, 0] += jnp.sum(a + mind)


def _vq_argmin(flat, W):
    return pl.pallas_call(
        _vq_tc_body,
        grid=(N // TT,),
        in_specs=[
            pl.BlockSpec((TT, D), lambda i: (i, 0)),
            pl.BlockSpec((K, D), lambda i: (0, 0)),
        ],
        out_specs=[
            pl.BlockSpec((TT,), lambda i: (i,)),
            pl.BlockSpec((1, 1), lambda i: (0, 0), memory_space=pltpu.SMEM),
        ],
        out_shape=[
            jax.ShapeDtypeStruct((N,), jnp.int32),
            jax.ShapeDtypeStruct((1, 1), jnp.float32),
        ],
    )(flat, W)


def _sc_gather(table, idx):
    info = plsc.get_sparse_core_info()
    nw = info.num_cores * info.num_subcores
    b_per_w = N // nw
    mesh = plsc.VectorSubcoreMesh(core_axis_name="c", subcore_axis_name="s")

    @functools.partial(
        pl.kernel, mesh=mesh,
        compiler_params=pltpu.CompilerParams(use_tc_tiling_on_sc=False),
        out_type=jax.ShapeDtypeStruct((N, D), jnp.float32),
        scratch_types=[
            pltpu.VMEM((b_per_w,), jnp.int32),
            pltpu.VMEM((b_per_w, D), jnp.float32),
            pltpu.SemaphoreType.DMA,
        ],
    )
    def k(idx_hbm, table_hbm, out_hbm, idx_v, rows_v, sem):
        wid = lax.axis_index("s") * info.num_cores + lax.axis_index("c")
        base = wid * b_per_w
        pltpu.sync_copy(idx_hbm.at[pl.ds(base, b_per_w)], idx_v)
        pltpu.async_copy(table_hbm.at[idx_v], rows_v, sem).wait()
        pltpu.sync_copy(rows_v, out_hbm.at[pl.ds(base, b_per_w)])

    return k(idx, table)


def kernel(inputs, W):
    x = jnp.transpose(inputs, (0, 2, 3, 1))
    flat = x.reshape(N, D)
    idx, loss_sum = _vq_argmin(flat, W)
    q_flat = _sc_gather(W, idx)
    quantized = q_flat.reshape(x.shape)
    encodings = idx.reshape(x.shape[0:3])
    loss = (1.0 + CC) * (loss_sum[0, 0] / (N * D))
    quantized_st = x + (quantized - x)
    out = jnp.transpose(quantized_st, (0, 3, 1, 2))
    nll = jnp.ones((1,), dtype=jnp.float32)
    return (out, encodings, loss, nll)
